# baseline (speedup 1.0000x reference)
import functools

import jax
import jax.numpy as jnp
from jax import lax
from jax.experimental import pallas as pl
from jax.experimental.pallas import tpu as pltpu
from jax.experimental.pallas import tpu_sc as plsc

SEQ = 200
DIM = 32
BATCH = 1024
VOCAB = 1000000
NW = 32
LANES = 16
BSUB = BATCH // 128


def _sc_body(idx_hbm, tok_hbm, pos_hbm, out_hbm, idx_v, pos_v, out_v, sem):
    cid = lax.axis_index("c")
    sid = lax.axis_index("s")
    w = sid * 2 + cid

    pltpu.sync_copy(pos_hbm, pos_v)

    def do_l(l):
        pltpu.sync_copy(idx_hbm.at[l], idx_v)
        copies = [
            pltpu.async_copy(tok_hbm.at[d].at[idx_v], out_v.at[d], sem)
            for d in range(DIM)
        ]
        for cp in copies:
            cp.wait()
        for d in range(DIM):
            p = plsc.load_gather(
                pos_v,
                [jnp.full((LANES,), d, jnp.int32),
                 jnp.full((LANES,), l, jnp.int32)],
            )

            def add_j(j, _):
                for k in range(8):
                    out_v[d, pl.ds(j * 128 + k * LANES, LANES)] += p
                return 0

            lax.fori_loop(0, BSUB, add_j, 0)
        pltpu.sync_copy(out_v, out_hbm.at[l])

    for k in range(6):
        do_l(w + NW * k)

    @pl.when(w < SEQ - 6 * NW)
    def _():
        do_l(w + NW * 6)


@jax.jit
def _run(idx3, tok_t, pos_t):
    mesh = plsc.VectorSubcoreMesh(core_axis_name="c", subcore_axis_name="s")
    return pl.kernel(
        _sc_body,
        out_type=jax.ShapeDtypeStruct((SEQ, DIM, BATCH), jnp.float32),
        mesh=mesh,
        scratch_types=[
            pltpu.VMEM((BATCH,), jnp.int32),
            pltpu.VMEM((DIM, SEQ), jnp.float32),
            pltpu.VMEM((DIM, BATCH), jnp.float32),
            pltpu.SemaphoreType.DMA,
        ],
        compiler_params=pltpu.CompilerParams(
            use_tc_tiling_on_sc=False, needs_layout_passes=False),
    )(idx3, tok_t, pos_t)


def kernel(inputs, token_table, pos_table):
    idx_t = inputs.astype(jnp.int32).T
    out = _run(idx_t, token_table.T, pos_table.T)
    return jnp.transpose(out, (2, 0, 1))


# speedup vs baseline: 1.0097x; 1.0097x over previous
import functools

import jax
import jax.numpy as jnp
from jax import lax
from jax.experimental import pallas as pl
from jax.experimental.pallas import tpu as pltpu
from jax.experimental.pallas import tpu_sc as plsc

SEQ = 200
DIM = 32
BATCH = 1024
VOCAB = 1000000
LANES = 16
NTILE = 16
D_PER_CORE = DIM // 2
MAX_L = 13
STAGE_CHUNK = 64000


def _sc_body(idx_hbm, tok_hbm, pos_hbm, out_hbm,
             idx_all, pos_v, row_v, row_spmem, sem):
    cid = lax.axis_index("c")
    sid = lax.axis_index("s")

    pltpu.sync_copy(pos_hbm, pos_v)

    n_l = jnp.where(sid < SEQ - 12 * NTILE, 13, 12)
    for j in range(MAX_L):
        @pl.when(jnp.int32(j) < n_l)
        def _():
            pltpu.sync_copy(idx_hbm.at[sid + NTILE * j], idx_all.at[j])

    chunk = jnp.where(sid < NTILE - 1, STAGE_CHUNK, VOCAB - 15 * STAGE_CHUNK)
    base = sid * STAGE_CHUNK

    def do_d(dd, _):
        d = cid * D_PER_CORE + dd
        pltpu.sync_copy(tok_hbm.at[d, pl.ds(base, chunk)],
                        row_spmem.at[pl.ds(base, chunk)])
        plsc.subcore_barrier()

        def do_l(j, _):
            l = sid + NTILE * j
            pltpu.async_copy(row_spmem.at[idx_all.at[j]], row_v, sem).wait()
            p = plsc.load_gather(
                pos_v,
                [jnp.full((LANES,), d, jnp.int32),
                 jnp.full((LANES,), l, jnp.int32)],
            )

            def add_k(k, _):
                row_v[pl.ds(k * LANES, LANES)] += p
                return 0

            lax.fori_loop(0, BATCH // LANES, add_k, 0)
            pltpu.sync_copy(row_v, out_hbm.at[l, d])
            return 0

        lax.fori_loop(0, n_l, do_l, 0)
        plsc.subcore_barrier()
        return 0

    lax.fori_loop(0, D_PER_CORE, do_d, 0)


@jax.jit
def _run(idx_t, tok_t, pos_t):
    mesh = plsc.VectorSubcoreMesh(core_axis_name="c", subcore_axis_name="s")
    return pl.kernel(
        _sc_body,
        out_type=jax.ShapeDtypeStruct((SEQ, DIM, BATCH), jnp.float32),
        mesh=mesh,
        scratch_types=[
            pltpu.VMEM((MAX_L, BATCH), jnp.int32),
            pltpu.VMEM((DIM, SEQ), jnp.float32),
            pltpu.VMEM((BATCH,), jnp.float32),
            pltpu.VMEM_SHARED((VOCAB,), jnp.float32),
            pltpu.SemaphoreType.DMA,
        ],
        compiler_params=pltpu.CompilerParams(
            use_tc_tiling_on_sc=False, needs_layout_passes=False),
    )(idx_t, tok_t, pos_t)


def kernel(inputs, token_table, pos_table):
    idx_t = inputs.astype(jnp.int32).T
    out = _run(idx_t, token_table.T, pos_table.T)
    return jnp.transpose(out, (2, 0, 1))


# speedup vs baseline: 4.7901x; 4.7440x over previous
import functools

import jax
import jax.numpy as jnp
from jax import lax
from jax.experimental import pallas as pl
from jax.experimental.pallas import tpu as pltpu
from jax.experimental.pallas import tpu_sc as plsc

SEQ = 200
DIM = 32
BATCH = 1024
VOCAB = 1000000
LANES = 16
NW = 32
OPAD = 1025


def _sc_body(idx_hbm, tok_hbm, pos_hbm, out_hbm,
             idx_v, pos_v, rows_v, out_v, sem):
    cid = lax.axis_index("c")
    sid = lax.axis_index("s")
    w = sid * 2 + cid

    pltpu.sync_copy(pos_hbm, pos_v)
    dlanes = lax.iota(jnp.int32, LANES)

    def do_l(l):
        pltpu.sync_copy(idx_hbm.at[l], idx_v)
        copies = [
            pltpu.async_copy(
                tok_hbm.at[idx_v.at[pl.ds(j * 128, 128)]],
                rows_v.at[pl.ds(j * 128, 128)],
                sem,
            )
            for j in range(BATCH // 128)
        ]
        for cp in copies:
            cp.wait()

        p0 = plsc.load_gather(pos_v, [dlanes, jnp.full((LANES,), l, jnp.int32)])
        p1 = plsc.load_gather(
            pos_v, [dlanes + LANES, jnp.full((LANES,), l, jnp.int32)])

        def tr_j(j, _):
            v0 = rows_v[j, pl.ds(0, LANES)] + p0
            v1 = rows_v[j, pl.ds(LANES, LANES)] + p1
            jb = jnp.full((LANES,), j, jnp.int32)
            plsc.store_scatter(out_v, [dlanes, jb], v0)
            plsc.store_scatter(out_v, [dlanes + LANES, jb], v1)
            return 0

        lax.fori_loop(0, BATCH, tr_j, 0)
        pltpu.sync_copy(out_v.at[:, pl.ds(0, BATCH)], out_hbm.at[l])

    for k in range(6):
        do_l(w + NW * k)

    @pl.when(w < SEQ - 6 * NW)
    def _():
        do_l(w + NW * 6)


@jax.jit
def _run(idx_t, tok, pos_t):
    mesh = plsc.VectorSubcoreMesh(core_axis_name="c", subcore_axis_name="s")
    return pl.kernel(
        _sc_body,
        out_type=jax.ShapeDtypeStruct((SEQ, DIM, BATCH), jnp.float32),
        mesh=mesh,
        scratch_types=[
            pltpu.VMEM((BATCH,), jnp.int32),
            pltpu.VMEM((DIM, SEQ), jnp.float32),
            pltpu.VMEM((BATCH, DIM), jnp.float32),
            pltpu.VMEM((DIM, OPAD), jnp.float32),
            pltpu.SemaphoreType.DMA,
        ],
        compiler_params=pltpu.CompilerParams(
            use_tc_tiling_on_sc=False, needs_layout_passes=False),
    )(idx_t, tok, pos_t)


def kernel(inputs, token_table, pos_table):
    idx_t = inputs.astype(jnp.int32).T
    out = _run(idx_t, token_table, pos_table.T)
    return jnp.transpose(out, (2, 0, 1))
